# baseline (speedup 1.0000x reference)
import functools

import jax
import jax.numpy as jnp
from jax import lax
from jax.experimental import pallas as pl
from jax.experimental.pallas import tpu as pltpu
from jax.experimental.pallas import tpu_sc as plsc

NUM_ENT_PAIRS = 100000
N_RELS = 64
ROW_PAD = 128
BATCH = 4096
LANES = 16
NC = 2
NS = 16
NW = NC * NS
BPW = BATCH // NW


def _sc_body(idx_hbm, kb_hbm, sig_hbm, idx_v, rows_v, sig_v, sem):
    wid = lax.axis_index("s") * NC + lax.axis_index("c")
    base = wid * BPW

    pltpu.sync_copy(idx_hbm.at[pl.ds(base, BPW)], idx_v)
    pltpu.async_copy(kb_hbm.at[idx_v], rows_v, sem).wait()

    def row(r, carry):
        for c in range(N_RELS // LANES):
            x = rows_v[r, pl.ds(c * LANES, LANES)]
            sig_v[r, pl.ds(c * LANES, LANES)] = 1.0 / (1.0 + jnp.exp(-x))
        return carry

    lax.fori_loop(0, BPW, row, 0)

    pltpu.sync_copy(sig_v, sig_hbm.at[pl.ds(base, BPW)])


def _tc_body(sig_ref, p_ref, probs_ref):
    probs_ref[...] = jnp.sum(sig_ref[...] * p_ref[...], axis=1)


@jax.jit
def _run(ent_pair, p_pad, kb128):
    mesh = plsc.VectorSubcoreMesh(
        core_axis_name="c", subcore_axis_name="s",
        num_cores=NC, num_subcores=NS)
    sig = pl.kernel(
        _sc_body,
        out_type=jax.ShapeDtypeStruct((BATCH, N_RELS), jnp.float32),
        mesh=mesh,
        scratch_types=(
            pltpu.VMEM((BPW,), jnp.int32),
            pltpu.VMEM((BPW, ROW_PAD), jnp.float32),
            pltpu.VMEM((BPW, N_RELS), jnp.float32),
            pltpu.SemaphoreType.DMA,
        ),
        compiler_params=pltpu.CompilerParams(use_tc_tiling_on_sc=False),
    )(ent_pair, kb128)

    probs = pl.pallas_call(
        _tc_body,
        out_shape=jax.ShapeDtypeStruct((BATCH,), jnp.float32),
    )(sig, p_pad)
    return sig, probs


def kernel(ent_pair, p_not_na, p_rel_not_na, reprs, kb_table):
    idx = ent_pair.astype(jnp.int32)
    kb128 = jnp.pad(kb_table, ((0, 0), (0, ROW_PAD - N_RELS)))
    p_pad = jnp.concatenate(
        [jnp.zeros((BATCH, 1), jnp.float32), p_rel_not_na], axis=1)
    ent_pair_rel, probs = _run(idx, p_pad, kb128)
    return (probs, p_not_na, p_rel_not_na, reprs, ent_pair_rel)
